# baseline (speedup 1.0000x reference)
import functools

import jax
import jax.numpy as jnp
from jax import lax
from jax.experimental import pallas as pl
from jax.experimental.pallas import tpu as pltpu
from jax.experimental.pallas import tpu_sc as plsc

NC = 2
NS = 16
L = 16
NW = NC * NS

BATCH = 16384
DIM = 64
NROWS = 1000000
BPW = BATCH // NW
WIN = NROWS // NW
SLABW = 31264
MAXOWN = 768
SCR = BATCH + NW
NBLK = 16
BLK = BATCH // NBLK


def _p1_body(ut, it, user, item, scr_u, scr_v,
             idxc, ocol, oeid, slab, ubt, rowb, sem, ssem):
    wid = lax.axis_index("s") * NC + lax.axis_index("c")
    lo = wid * WIN
    alo = pl.multiple_of(jnp.minimum((lo >> 3) << 3, NROWS - SLABW), 8)
    lane = lax.iota(jnp.int32, L)

    for t, (tab, idx_hbm, scr) in enumerate(
            ((ut, user, scr_u), (it, item, scr_v))):
        def blkscan(blk, cnt):
            pltpu.sync_copy(idx_hbm.at[pl.ds(blk * BLK, BLK)], idxc)

            def scan(v, cnt):
                c = idxc[pl.ds(v * L, L)]
                rel = c - lo
                m = (rel >= 0) & (rel < WIN)
                e = blk * BLK + v * L + lane
                plsc.store_compressed(ocol.at[pl.ds(cnt, L)], c, mask=m)
                plsc.store_compressed(oeid.at[pl.ds(cnt, L)], e, mask=m)
                return cnt + jnp.sum(m.astype(jnp.int32))

            return lax.fori_loop(0, BLK // L, scan, cnt)

        cnt = lax.fori_loop(0, NBLK, blkscan, jnp.int32(0))

        ocol[pl.ds(cnt, L)] = jnp.full((L,), lo, jnp.int32)
        oeid[pl.ds(cnt, L)] = jnp.full((L,), BATCH + wid, jnp.int32)
        trips = (cnt + L - 1) >> 4

        def slab_cp(d, b):
            return pltpu.make_async_copy(
                tab.at[d, pl.ds(alo, SLABW)], slab.at[b], sem)

        slab_cp(0, 0).start()

        def feat(d, carry):
            @pl.when(d + 1 < DIM)
            def _():
                slab_cp(d + 1, (d + 1) & 1).start()

            slab_cp(d, d & 1).wait()

            def gat(k, carry):
                c = ocol[pl.ds(k * L, L)] - alo
                ubt[d, pl.ds(k * L, L)] = plsc.load_gather(slab.at[d & 1], [c])
                return carry

            return lax.fori_loop(0, trips, gat, carry)

        lax.fori_loop(0, DIM, feat, 0)

        def put(k, carry):
            for i in range(L):
                ev = jnp.full((L,), k * L + i, jnp.int32)
                for q in range(DIM // L):
                    rowb[i, pl.ds(q * L, L)] = plsc.load_gather(
                        ubt, [q * L + lane, ev])
            pltpu.make_async_copy(
                rowb, scr.at[oeid.at[pl.ds(k * L, L)]], ssem).start()
            pltpu.make_async_copy(
                rowb, scr.at[oeid.at[pl.ds(k * L, L)]], ssem).wait()
            return carry

        lax.fori_loop(0, trips, put, 0)


def _p2_body(scr_u, scr_v, out, ubuf, vbuf, out_v):
    wid = lax.axis_index("s") * NC + lax.axis_index("c")
    base = wid * BPW
    pltpu.sync_copy(scr_u.at[pl.ds(base, BPW), :], ubuf)
    pltpu.sync_copy(scr_v.at[pl.ds(base, BPW), :], vbuf)

    lane = lax.iota(jnp.int32, L)

    def group(g, carry):
        def elem(k, r):
            e = g * L + k
            p = (ubuf[e, pl.ds(0, 16)] * vbuf[e, pl.ds(0, 16)]
                 + ubuf[e, pl.ds(16, 16)] * vbuf[e, pl.ds(16, 16)]
                 + ubuf[e, pl.ds(32, 16)] * vbuf[e, pl.ds(32, 16)]
                 + ubuf[e, pl.ds(48, 16)] * vbuf[e, pl.ds(48, 16)])
            return jnp.where(lane == k, jnp.sum(p), r)

        r = lax.fori_loop(0, L, elem, jnp.zeros((L,), jnp.float32))
        out_v[pl.ds(g * L, L)] = 1.0 / (1.0 + jnp.exp(-r))
        return carry

    lax.fori_loop(0, BPW // L, group, 0)
    pltpu.sync_copy(out_v, out.at[pl.ds(base, BPW)])


_mesh = plsc.VectorSubcoreMesh(core_axis_name="c", subcore_axis_name="s")
_params = pltpu.CompilerParams(
    needs_layout_passes=False, use_tc_tiling_on_sc=False)

_p1 = functools.partial(
    pl.kernel,
    out_type=(jax.ShapeDtypeStruct((SCR, DIM), jnp.float32),
              jax.ShapeDtypeStruct((SCR, DIM), jnp.float32)),
    mesh=_mesh,
    scratch_types=[
        pltpu.VMEM((BLK,), jnp.int32),
        pltpu.VMEM((MAXOWN + L,), jnp.int32),
        pltpu.VMEM((MAXOWN + L,), jnp.int32),
        pltpu.VMEM((2, SLABW), jnp.float32),
        pltpu.VMEM((DIM, MAXOWN), jnp.float32),
        pltpu.VMEM((L, DIM), jnp.float32),
        pltpu.SemaphoreType.DMA,
        pltpu.SemaphoreType.DMA,
    ],
    compiler_params=_params,
)(_p1_body)

_p2 = functools.partial(
    pl.kernel,
    out_type=jax.ShapeDtypeStruct((BATCH,), jnp.float32),
    mesh=_mesh,
    scratch_types=[
        pltpu.VMEM((BPW, DIM), jnp.float32),
        pltpu.VMEM((BPW, DIM), jnp.float32),
        pltpu.VMEM((BPW,), jnp.float32),
    ],
    compiler_params=_params,
)(_p2_body)


def kernel(user_table, item_table, user, item):
    if True:
        scr_u, scr_v = _p1(user_table.T, item_table.T,
                           user.astype(jnp.int32), item.astype(jnp.int32))
        return scr_u[:BATCH, 0]
    scr_u, scr_v = _p1(user_table.T, item_table.T,
                       user.astype(jnp.int32), item.astype(jnp.int32))
    return _p2(scr_u, scr_v)


# speedup vs baseline: 1.0011x; 1.0011x over previous
import functools

import jax
import jax.numpy as jnp
from jax import lax
from jax.experimental import pallas as pl
from jax.experimental.pallas import tpu as pltpu
from jax.experimental.pallas import tpu_sc as plsc

NC = 2
NS = 16
L = 16
NW = NC * NS

BATCH = 16384
DIM = 64
NROWS = 1000000
BPW = BATCH // NW
WIN = NROWS // NW
TROW = (NROWS * DIM) // 128
SROWS = 264
MAXOWN = 768
SCR = BATCH + NW
NBLK = 16
BLK = BATCH // NBLK


def _p1_body(ut, it, user, item, scr_u, scr_v,
             idxc, ocol, oeid, slab, ubt, rowb, sem, ssem):
    wid = lax.axis_index("s") * NC + lax.axis_index("c")
    lo = wid * WIN
    lane = lax.iota(jnp.int32, L)

    def arow_of(d):
        r = ((d * NROWS + lo) >> 7) & ~7
        return pl.multiple_of(jnp.minimum(r, TROW - SROWS), 8)

    for t, (tab, idx_hbm, scr) in enumerate(
            ((ut, user, scr_u), (it, item, scr_v))):
        def blkscan(blk, cnt):
            pltpu.sync_copy(idx_hbm.at[pl.ds(blk * BLK, BLK)], idxc)

            def scan(v, cnt):
                c = idxc[pl.ds(v * L, L)]
                rel = c - lo
                m = (rel >= 0) & (rel < WIN)
                e = blk * BLK + v * L + lane
                plsc.store_compressed(ocol.at[pl.ds(cnt, L)], c, mask=m)
                plsc.store_compressed(oeid.at[pl.ds(cnt, L)], e, mask=m)
                return cnt + jnp.sum(m.astype(jnp.int32))

            return lax.fori_loop(0, BLK // L, scan, cnt)

        cnt = lax.fori_loop(0, NBLK, blkscan, jnp.int32(0))

        ocol[pl.ds(cnt, L)] = jnp.full((L,), lo, jnp.int32)
        oeid[pl.ds(cnt, L)] = jnp.full((L,), BATCH + wid, jnp.int32)
        trips = (cnt + L - 1) >> 4

        def slab_cp(d, b):
            return pltpu.make_async_copy(
                tab.at[pl.ds(arow_of(d), SROWS), :], slab.at[b], sem)

        slab_cp(0, 0).start()

        def feat(d, carry):
            @pl.when(d + 1 < DIM)
            def _():
                slab_cp(d + 1, (d + 1) & 1).start()

            slab_cp(d, d & 1).wait()
            base = d * NROWS - (arow_of(d) << 7)

            def gat(k, carry):
                c = ocol[pl.ds(k * L, L)] + base
                ubt[d, pl.ds(k * L, L)] = plsc.load_gather(
                    slab.at[d & 1], [c >> 7, c & 127])
                return carry

            return lax.fori_loop(0, trips, gat, carry)

        lax.fori_loop(0, DIM, feat, 0)

        def rowput(k, b):
            for i in range(L):
                ev = jnp.full((L,), k * L + i, jnp.int32)
                for q in range(DIM // L):
                    rowb[b, i, pl.ds(q * L, L)] = plsc.load_gather(
                        ubt, [q * L + lane, ev])
            pltpu.make_async_copy(
                rowb.at[b], scr.at[oeid.at[pl.ds(k * L, L)]], ssem).start()

        def put(k, carry):
            rowput(k, k & 1)

            @pl.when(k > 0)
            def _():
                pltpu.make_async_copy(
                    rowb.at[(k - 1) & 1],
                    scr.at[oeid.at[pl.ds((k - 1) * L, L)]], ssem).wait()

            return carry

        lax.fori_loop(0, trips, put, 0)
        pltpu.make_async_copy(
            rowb.at[(trips - 1) & 1],
            scr.at[oeid.at[pl.ds((trips - 1) * L, L)]], ssem).wait()


def _p2_body(scr_u, scr_v, out, ubuf, vbuf, out_v):
    wid = lax.axis_index("s") * NC + lax.axis_index("c")
    base = wid * BPW
    pltpu.sync_copy(scr_u.at[pl.ds(base, BPW), :], ubuf)
    pltpu.sync_copy(scr_v.at[pl.ds(base, BPW), :], vbuf)

    lane = lax.iota(jnp.int32, L)

    def group(g, carry):
        def elem(k, r):
            e = g * L + k
            p = (ubuf[e, pl.ds(0, 16)] * vbuf[e, pl.ds(0, 16)]
                 + ubuf[e, pl.ds(16, 16)] * vbuf[e, pl.ds(16, 16)]
                 + ubuf[e, pl.ds(32, 16)] * vbuf[e, pl.ds(32, 16)]
                 + ubuf[e, pl.ds(48, 16)] * vbuf[e, pl.ds(48, 16)])
            return jnp.where(lane == k, jnp.sum(p), r)

        r = lax.fori_loop(0, L, elem, jnp.zeros((L,), jnp.float32))
        out_v[pl.ds(g * L, L)] = 1.0 / (1.0 + jnp.exp(-r))
        return carry

    lax.fori_loop(0, BPW // L, group, 0)
    pltpu.sync_copy(out_v, out.at[pl.ds(base, BPW)])


_mesh = plsc.VectorSubcoreMesh(core_axis_name="c", subcore_axis_name="s")
_params = pltpu.CompilerParams(
    needs_layout_passes=False, use_tc_tiling_on_sc=False)

_p1 = functools.partial(
    pl.kernel,
    out_type=(jax.ShapeDtypeStruct((SCR, DIM), jnp.float32),
              jax.ShapeDtypeStruct((SCR, DIM), jnp.float32)),
    mesh=_mesh,
    scratch_types=[
        pltpu.VMEM((BLK,), jnp.int32),
        pltpu.VMEM((MAXOWN + L,), jnp.int32),
        pltpu.VMEM((MAXOWN + L,), jnp.int32),
        pltpu.VMEM((2, SROWS, 128), jnp.float32),
        pltpu.VMEM((DIM, MAXOWN), jnp.float32),
        pltpu.VMEM((2, L, DIM), jnp.float32),
        pltpu.SemaphoreType.DMA,
        pltpu.SemaphoreType.DMA,
    ],
    compiler_params=_params,
)(_p1_body)

_p2 = functools.partial(
    pl.kernel,
    out_type=jax.ShapeDtypeStruct((BATCH,), jnp.float32),
    mesh=_mesh,
    scratch_types=[
        pltpu.VMEM((BPW, DIM), jnp.float32),
        pltpu.VMEM((BPW, DIM), jnp.float32),
        pltpu.VMEM((BPW,), jnp.float32),
    ],
    compiler_params=_params,
)(_p2_body)


def kernel(user_table, item_table, user, item):
    tu = user_table.T.reshape(TROW, 128)
    ti = item_table.T.reshape(TROW, 128)
    scr_u, scr_v = _p1(tu, ti,
                       user.astype(jnp.int32), item.astype(jnp.int32))
    return _p2(scr_u, scr_v)


# speedup vs baseline: 31.6292x; 31.5940x over previous
import functools

import jax
import jax.numpy as jnp
from jax import lax
from jax.experimental import pallas as pl
from jax.experimental.pallas import tpu as pltpu
from jax.experimental.pallas import tpu_sc as plsc

NC = 2
NS = 16
L = 16
NW = NC * NS

BATCH = 16384
DIM = 64
NROWS = 1000000
BPW = BATCH // NW
WIN = NROWS // NW
CW = 2304
NCHK = (WIN + 127 + CW - 1) // CW
MAXOWN = 768
SCR = BATCH + NW
NBLK = 16
BLK = BATCH // NBLK


def _p1_body(ut, it, user, item, scr_u, scr_v,
             idxc, ocol, oeid, mcol, mpos, slab, ubt, rowb, sem, ssem):
    wid = lax.axis_index("s") * NC + lax.axis_index("c")
    lo = wid * WIN
    a0 = (lo >> 7) << 7
    lane = lax.iota(jnp.int32, L)

    def acol_of(j):
        return pl.multiple_of(jnp.minimum(a0 + j * CW, NROWS - CW), 128)

    for t, (tab, idx_hbm, scr) in enumerate(
            ((ut, user, scr_u), (it, item, scr_v))):
        def blkscan(blk, cnt):
            pltpu.sync_copy(idx_hbm.at[pl.ds(blk * BLK, BLK)], idxc)

            def scan(v, cnt):
                c = idxc[pl.ds(v * L, L)]
                rel = c - lo
                m = (rel >= 0) & (rel < WIN)
                e = blk * BLK + v * L + lane
                plsc.store_compressed(ocol.at[pl.ds(cnt, L)], c, mask=m)
                plsc.store_compressed(oeid.at[pl.ds(cnt, L)], e, mask=m)
                return cnt + jnp.sum(m.astype(jnp.int32))

            return lax.fori_loop(0, BLK // L, scan, cnt)

        cnt = lax.fori_loop(0, NBLK, blkscan, jnp.int32(0))

        ocol[pl.ds(cnt, L)] = jnp.full((L,), lo, jnp.int32)
        oeid[pl.ds(cnt, L)] = jnp.full((L,), BATCH + wid, jnp.int32)
        trips = (cnt + L - 1) >> 4

        def slab_cp(j, b, buf):
            return pltpu.make_async_copy(
                tab.at[pl.ds(b * 8, 8), pl.ds(acol_of(j), CW)],
                slab.at[buf], sem)

        slab_cp(0, 0, 0).start()

        def chunk(j, carry):
            sel_lo = a0 + j * CW
            acol = acol_of(j)

            def msel(k, mcnt):
                c = ocol[pl.ds(k * L, L)]
                m = (c >= sel_lo) & (c < sel_lo + CW)
                pos = k * L + lane
                plsc.store_compressed(mcol.at[pl.ds(mcnt, L)], c, mask=m)
                plsc.store_compressed(mpos.at[pl.ds(mcnt, L)], pos, mask=m)
                return mcnt + jnp.sum(m.astype(jnp.int32))

            mcnt = lax.fori_loop(0, trips, msel, jnp.int32(0))
            mcol[pl.ds(mcnt, L)] = jnp.full((L,), acol, jnp.int32)
            mpos[pl.ds(mcnt, L)] = jnp.full((L,), MAXOWN, jnp.int32)
            mtrips = (mcnt + L - 1) >> 4

            for b in range(8):
                if b + 1 < 8:
                    slab_cp(j, b + 1, (b + 1) & 1).start()
                else:
                    @pl.when(j + 1 < NCHK)
                    def _():
                        slab_cp(j + 1, 0, 0).start()

                slab_cp(j, b, b & 1).wait()

                def gat(k, carry):
                    cl = mcol[pl.ds(k * L, L)] - acol
                    p = mpos[pl.ds(k * L, L)]
                    for r in range(8):
                        v = plsc.load_gather(
                            slab.at[b & 1], [jnp.full((L,), r, jnp.int32), cl])
                        plsc.store_scatter(
                            ubt, [jnp.full((L,), b * 8 + r, jnp.int32), p], v)
                    return carry

                lax.fori_loop(0, mtrips, gat, 0)

            return carry

        lax.fori_loop(0, NCHK, chunk, 0)

        def rowput(k, b):
            for i in range(L):
                ev = jnp.full((L,), k * L + i, jnp.int32)
                for q in range(DIM // L):
                    rowb[b, i, pl.ds(q * L, L)] = plsc.load_gather(
                        ubt, [q * L + lane, ev])
            pltpu.make_async_copy(
                rowb.at[b], scr.at[oeid.at[pl.ds(k * L, L)]], ssem).start()

        def put(k, carry):
            rowput(k, k & 1)

            @pl.when(k > 0)
            def _():
                pltpu.make_async_copy(
                    rowb.at[(k - 1) & 1],
                    scr.at[oeid.at[pl.ds((k - 1) * L, L)]], ssem).wait()

            return carry

        lax.fori_loop(0, trips, put, 0)
        pltpu.make_async_copy(
            rowb.at[(trips - 1) & 1],
            scr.at[oeid.at[pl.ds((trips - 1) * L, L)]], ssem).wait()


def _p2_body(scr_u, scr_v, out, ubuf, vbuf, out_v):
    wid = lax.axis_index("s") * NC + lax.axis_index("c")
    base = wid * BPW
    lane = lax.iota(jnp.int32, L)
    HALF = BPW // 2

    for h in range(2):
        hb = base + h * HALF
        pltpu.sync_copy(scr_u.at[pl.ds(hb, HALF), :], ubuf)
        pltpu.sync_copy(scr_v.at[pl.ds(hb, HALF), :], vbuf)

        def group(g, carry):
            def elem(k, r):
                e = g * L + k
                p = (ubuf[e, pl.ds(0, 16)] * vbuf[e, pl.ds(0, 16)]
                     + ubuf[e, pl.ds(16, 16)] * vbuf[e, pl.ds(16, 16)]
                     + ubuf[e, pl.ds(32, 16)] * vbuf[e, pl.ds(32, 16)]
                     + ubuf[e, pl.ds(48, 16)] * vbuf[e, pl.ds(48, 16)])
                return jnp.where(lane == k, jnp.sum(p), r)

            r = lax.fori_loop(0, L, elem, jnp.zeros((L,), jnp.float32))
            out_v[pl.ds(h * HALF + g * L, L)] = 1.0 / (1.0 + jnp.exp(-r))
            return carry

        lax.fori_loop(0, HALF // L, group, 0)

    pltpu.sync_copy(out_v, out.at[pl.ds(base, BPW)])


_mesh = plsc.VectorSubcoreMesh(core_axis_name="c", subcore_axis_name="s")
_params = pltpu.CompilerParams(
    needs_layout_passes=False, use_tc_tiling_on_sc=True)

_p1 = functools.partial(
    pl.kernel,
    out_type=(jax.ShapeDtypeStruct((SCR, 128), jnp.float32),
              jax.ShapeDtypeStruct((SCR, 128), jnp.float32)),
    mesh=_mesh,
    scratch_types=[
        pltpu.VMEM((BLK,), jnp.int32),
        pltpu.VMEM((MAXOWN + L,), jnp.int32),
        pltpu.VMEM((MAXOWN + L,), jnp.int32),
        pltpu.VMEM((MAXOWN + L,), jnp.int32),
        pltpu.VMEM((MAXOWN + L,), jnp.int32),
        pltpu.VMEM((2, 8, CW), jnp.float32),
        pltpu.VMEM((DIM, MAXOWN + L), jnp.float32),
        pltpu.VMEM((2, L, 128), jnp.float32),
        pltpu.SemaphoreType.DMA,
        pltpu.SemaphoreType.DMA,
    ],
    compiler_params=_params,
)(_p1_body)

_p2 = functools.partial(
    pl.kernel,
    out_type=jax.ShapeDtypeStruct((BATCH,), jnp.float32),
    mesh=_mesh,
    scratch_types=[
        pltpu.VMEM((BPW // 2, 128), jnp.float32),
        pltpu.VMEM((BPW // 2, 128), jnp.float32),
        pltpu.VMEM((BPW,), jnp.float32),
    ],
    compiler_params=_params,
)(_p2_body)


def kernel(user_table, item_table, user, item):
    scr_u, scr_v = _p1(user_table.T, item_table.T,
                       user.astype(jnp.int32), item.astype(jnp.int32))
    return _p2(scr_u, scr_v)
